# Initial kernel scaffold; baseline (speedup 1.0000x reference)
import functools

import jax
import jax.numpy as jnp
from jax import lax
from jax.experimental import pallas as pl
from jax.experimental.pallas import tpu as pltpu
from jax.experimental.pallas import tpu_sc as plsc

LANES = 16


def kernel(target, context, target_table, context_table):
    if target.ndim == 2:
        target = jnp.squeeze(target, axis=1)
    target = target.astype(jnp.int32)
    context = context.astype(jnp.int32)

    B = target.shape[0]
    C = context.shape[1]
    E = target_table.shape[1]
    EV = E // LANES

    info = plsc.get_sparse_core_info()
    NW = info.num_cores * info.num_subcores
    PB = 128
    nb = B // NW
    NP = nb // PB

    tgt2d = target.reshape(B // PB, PB)
    ctx2d = context.reshape(B * C // PB, PB)

    mesh = plsc.VectorSubcoreMesh(core_axis_name="c", subcore_axis_name="s")

    @functools.partial(
        pl.kernel,
        out_type=jax.ShapeDtypeStruct((B * C,), jnp.float32),
        mesh=mesh,
        scratch_types=[
            pltpu.VMEM((NP, PB), jnp.int32),
            pltpu.VMEM((NP * C, PB), jnp.int32),
            pltpu.VMEM((PB, E), jnp.float32),
            pltpu.VMEM((PB * C, E), jnp.float32),
            pltpu.VMEM((PB * C,), jnp.float32),
            pltpu.SemaphoreType.DMA,
        ],
    )
    def sc_kernel(tgt_idx_hbm, ctx_idx_hbm, tgt_tab, ctx_tab, out_hbm,
                  tgt_idx, ctx_idx, w_rows, c_rows, out_v, sem):
        cid = lax.axis_index("c")
        sid = lax.axis_index("s")
        wid = sid * info.num_cores + cid

        pltpu.sync_copy(tgt_idx_hbm.at[pl.ds(wid * NP, NP)], tgt_idx)
        pltpu.sync_copy(ctx_idx_hbm.at[pl.ds(wid * NP * C, NP * C)], ctx_idx)

        def phase(p, carry):
            cps = [pltpu.async_copy(tgt_tab.at[tgt_idx.at[p]], w_rows, sem)]
            for r in range(C):
                cps.append(pltpu.async_copy(
                    ctx_tab.at[ctx_idx.at[p * C + r]],
                    c_rows.at[pl.ds(r * PB, PB)], sem))
            for cp in cps:
                cp.wait()

            def body(b, acc_carry):
                w = [w_rows[b, pl.ds(LANES * j, LANES)] for j in range(EV)]
                for c in range(C):
                    row = b * C + c
                    acc = w[0] * c_rows[row, pl.ds(0, LANES)]
                    for j in range(1, EV):
                        acc = acc + w[j] * c_rows[row, pl.ds(LANES * j, LANES)]
                    out_v[row] = jnp.sum(acc)
                return acc_carry

            lax.fori_loop(0, PB, body, 0)
            pltpu.sync_copy(
                out_v, out_hbm.at[pl.ds((wid * NP + p) * PB * C, PB * C)])
            return carry

        lax.fori_loop(0, NP, phase, 0)

    out = sc_kernel(tgt2d, ctx2d, target_table, context_table)
    return out.reshape(B, C)


# speedup vs baseline: 10.3400x; 10.3400x over previous
import functools

import jax
import jax.numpy as jnp
from jax import lax
from jax.experimental import pallas as pl
from jax.experimental.pallas import tpu as pltpu
from jax.experimental.pallas import tpu_sc as plsc

LANES = 16


def kernel(target, context, target_table, context_table):
    if target.ndim == 2:
        target = jnp.squeeze(target, axis=1)
    target = target.astype(jnp.int32)
    context = context.astype(jnp.int32)

    B = target.shape[0]
    C = context.shape[1]
    E = target_table.shape[1]
    EV = E // LANES

    info = plsc.get_sparse_core_info()
    NW = info.num_cores * info.num_subcores
    PB = 128
    nb = B // NW
    NP = nb // PB

    ctx_flat = context.reshape(B * C)

    mesh = plsc.VectorSubcoreMesh(core_axis_name="c", subcore_axis_name="s")

    @functools.partial(
        pl.kernel,
        out_type=jax.ShapeDtypeStruct((B * C,), jnp.float32),
        mesh=mesh,
        compiler_params=pltpu.CompilerParams(needs_layout_passes=False),
        scratch_types=[
            pltpu.VMEM((nb,), jnp.int32),
            pltpu.VMEM((nb * C,), jnp.int32),
            pltpu.VMEM((PB, E), jnp.float32),
            pltpu.VMEM((PB * C, E), jnp.float32),
            pltpu.VMEM((PB * C,), jnp.float32),
            pltpu.VMEM((C * LANES, LANES), jnp.float32),
            pltpu.SemaphoreType.DMA,
        ],
    )
    def sc_kernel(tgt_idx_hbm, ctx_idx_hbm, tgt_tab, ctx_tab, out_hbm,
                  tgt_idx, ctx_idx, w_rows, c_rows, out_v, acc_buf, sem):
        cid = lax.axis_index("c")
        sid = lax.axis_index("s")
        wid = sid * info.num_cores + cid

        pltpu.sync_copy(tgt_idx_hbm.at[pl.ds(wid * nb, nb)], tgt_idx)
        pltpu.sync_copy(ctx_idx_hbm.at[pl.ds(wid * nb * C, nb * C)], ctx_idx)

        def phase(p, carry):
            cps = [pltpu.async_copy(
                tgt_tab.at[tgt_idx.at[pl.ds(p * PB, PB)]], w_rows, sem)]
            for r in range(C):
                cps.append(pltpu.async_copy(
                    ctx_tab.at[ctx_idx.at[pl.ds(p * PB * C + r * PB, PB)]],
                    c_rows.at[pl.ds(r * PB, PB)], sem))
            for cp in cps:
                cp.wait()

            lane = lax.iota(jnp.int32, LANES)

            def splat(v):
                return jnp.full((LANES,), v, jnp.int32)

            def body(g, acc_carry):
                for i in range(LANES):
                    b = g * LANES + i
                    w = [w_rows[b, pl.ds(LANES * j, LANES)] for j in range(EV)]
                    for c in range(C):
                        row = b * C + c
                        acc = w[0] * c_rows[row, pl.ds(0, LANES)]
                        for j in range(1, EV):
                            acc = acc + w[j] * c_rows[row, pl.ds(LANES * j, LANES)]
                        acc_buf[c * LANES + i, :] = acc
                for c in range(C):
                    rows_idx = splat(c * LANES) + lane
                    res = plsc.load_gather(acc_buf, [rows_idx, splat(0)])
                    for j in range(1, LANES):
                        res = res + plsc.load_gather(acc_buf, [rows_idx, splat(j)])
                    idx = g * (LANES * C) + lane * C + c
                    plsc.store_scatter(out_v, [idx], res)
                return acc_carry

            lax.fori_loop(0, PB // LANES, body, 0)
            pltpu.sync_copy(
                out_v, out_hbm.at[pl.ds((wid * NP + p) * PB * C, PB * C)])
            return carry

        lax.fori_loop(0, NP, phase, 0)

    out = sc_kernel(target, ctx_flat, target_table, context_table)
    return out.reshape(B, C)
